# Initial kernel scaffold; baseline (speedup 1.0000x reference)
import jax
import jax.numpy as jnp
from jax.experimental import pallas as pl


def kernel(x, weight, gamma, beta):
    raise NotImplementedError("write your pallas kernel here")


# speedup vs baseline: 2.3830x; 2.3830x over previous
import functools

import jax
import jax.numpy as jnp
from jax.experimental import pallas as pl
from jax.experimental.pallas import tpu as pltpu

_CI = 64
_CO = 128
_S = 24
_K = 3
_KCAT = _K * _CI
_BD = 6
_NDB = _S // _BD


def _p1_conv_stats(x_ref, w_ref, conv_ref, stats_ref, xcat):
    j = pl.program_id(1)

    @pl.when(j == 0)
    def _build():
        xcat[...] = jnp.zeros_like(xcat)
        xcat[1:_S + 1, 1:_S + 1, 1:_S, 0:_CI] = x_ref[:, :, 0:_S - 1, :]
        xcat[1:_S + 1, 1:_S + 1, :, _CI:2 * _CI] = x_ref[:, :, :, :]
        xcat[1:_S + 1, 1:_S + 1, 0:_S - 1, 2 * _CI:3 * _CI] = x_ref[:, :, 1:_S, :]
        stats_ref[...] = jnp.zeros_like(stats_ref)

    d0 = j * _BD
    acc = jnp.zeros((_BD * _S * _S, _CO), jnp.float32)
    for a in range(_K):
        for b in range(_K):
            lhs = xcat[pl.ds(d0 + a, _BD), pl.ds(b, _S), :, :]
            acc = acc + jnp.dot(lhs.reshape(_BD * _S * _S, _KCAT),
                                w_ref[_K * a + b],
                                preferred_element_type=jnp.float32)
    conv_ref[...] = acc.astype(jnp.bfloat16)
    stats_ref[0:1, :] += jnp.sum(acc, axis=0, keepdims=True)
    stats_ref[1:2, :] += jnp.sum(acc * acc, axis=0, keepdims=True)


def _p2_bn_relu(conv_ref, scale_ref, shift_ref, o_ref):
    y = conv_ref[...].astype(jnp.float32) * scale_ref[...] + shift_ref[...]
    o_ref[...] = jnp.maximum(y, 0.0)


def kernel(x, weight, gamma, beta):
    N = x.shape[0]
    eps = 1e-5
    P = _S * _S * _S

    xl = jnp.transpose(x, (0, 2, 3, 4, 1)).astype(jnp.bfloat16)

    wt = jnp.transpose(weight, (2, 3, 4, 1, 0))
    wt = wt.reshape(_K * _K, _KCAT, _CO).astype(jnp.bfloat16)

    conv, stats = pl.pallas_call(
        _p1_conv_stats,
        out_shape=[
            jax.ShapeDtypeStruct((N, P, _CO), jnp.bfloat16),
            jax.ShapeDtypeStruct((N, 2, _CO), jnp.float32),
        ],
        grid=(N, _NDB),
        in_specs=[
            pl.BlockSpec((None, _S, _S, _S, _CI), lambda n, j: (n, 0, 0, 0, 0)),
            pl.BlockSpec((_K * _K, _KCAT, _CO), lambda n, j: (0, 0, 0)),
        ],
        out_specs=[
            pl.BlockSpec((None, _BD * _S * _S, _CO), lambda n, j: (n, j, 0)),
            pl.BlockSpec((None, 2, _CO), lambda n, j: (n, 0, 0)),
        ],
        scratch_shapes=[
            pltpu.VMEM((_S + 2, _S + 2, _S, _KCAT), jnp.bfloat16),
        ],
        compiler_params=pltpu.CompilerParams(
            dimension_semantics=("parallel", "arbitrary")),
    )(xl, wt)

    M = N * P
    sums = jnp.sum(stats, axis=0)
    mean = sums[0] / M
    var = sums[1] / M - mean * mean
    scale = gamma.astype(jnp.float32) * jax.lax.rsqrt(var + eps)
    shift = beta.astype(jnp.float32) - mean * scale

    out_flat = pl.pallas_call(
        _p2_bn_relu,
        out_shape=jax.ShapeDtypeStruct((N, P, _CO), jnp.float32),
        grid=(N, _NDB),
        in_specs=[
            pl.BlockSpec((None, _BD * _S * _S, _CO), lambda n, j: (n, j, 0)),
            pl.BlockSpec((1, _CO), lambda n, j: (0, 0)),
            pl.BlockSpec((1, _CO), lambda n, j: (0, 0)),
        ],
        out_specs=pl.BlockSpec((None, _BD * _S * _S, _CO), lambda n, j: (n, j, 0)),
        compiler_params=pltpu.CompilerParams(
            dimension_semantics=("parallel", "parallel")),
    )(conv, scale.reshape(1, _CO), shift.reshape(1, _CO))

    out = out_flat.reshape(N, _S, _S, _S, _CO)
    return jnp.transpose(out, (0, 4, 1, 2, 3))
